# baseline (speedup 1.0000x reference)
import functools

import jax
import jax.numpy as jnp
from jax import lax
from jax.experimental import pallas as pl
from jax.experimental.pallas import tpu as pltpu
from jax.experimental.pallas import tpu_sc as plsc

NUM_CORES = 2
NUM_SUBCORES = 16
NUM_WORKERS = NUM_CORES * NUM_SUBCORES
CHUNK = 512


def _gather_kernel(total, d):
    b_per_w = total // NUM_WORKERS
    nchunks = b_per_w // CHUNK
    mesh = plsc.VectorSubcoreMesh(core_axis_name="c", subcore_axis_name="s")

    @functools.partial(
        pl.kernel,
        mesh=mesh,
        out_type=jax.ShapeDtypeStruct((total, d), jnp.float32),
        scratch_types=[
            pltpu.VMEM((CHUNK,), jnp.int32),
            pltpu.VMEM((CHUNK, d), jnp.float32),
            pltpu.SemaphoreType.DMA,
        ],
        compiler_params=pltpu.CompilerParams(use_tc_tiling_on_sc=False),
    )
    def k(idx_hbm, table_hbm, out_hbm, idx_v, rows_v, sem):
        wid = lax.axis_index("s") * NUM_CORES + lax.axis_index("c")
        base = wid * b_per_w

        def body(c, carry):
            off = base + c * CHUNK
            pltpu.sync_copy(idx_hbm.at[pl.ds(off, CHUNK)], idx_v)
            pltpu.async_copy(table_hbm.at[idx_v], rows_v, sem).wait()
            pltpu.sync_copy(rows_v, out_hbm.at[pl.ds(off, CHUNK)])
            return carry

        lax.fori_loop(0, nchunks, body, 0)

    return k


def kernel(x, table):
    b0, b1 = x.shape
    total = b0 * b1
    xf = x.reshape(total).astype(jnp.int32)
    out = _gather_kernel(total, table.shape[1])(xf, table)
    return out.reshape(b0, b1, table.shape[1])


# speedup vs baseline: 1.0354x; 1.0354x over previous
import functools

import jax
import jax.numpy as jnp
from jax import lax
from jax.experimental import pallas as pl
from jax.experimental.pallas import tpu as pltpu
from jax.experimental.pallas import tpu_sc as plsc

NUM_CORES = 2
NUM_SUBCORES = 16
NUM_WORKERS = NUM_CORES * NUM_SUBCORES
CHUNK = 512


def _gather_kernel(total, d):
    b_per_w = total // NUM_WORKERS
    nchunks = b_per_w // CHUNK
    mesh = plsc.VectorSubcoreMesh(core_axis_name="c", subcore_axis_name="s")

    assert nchunks % 2 == 0

    @functools.partial(
        pl.kernel,
        mesh=mesh,
        out_type=jax.ShapeDtypeStruct((total, d), jnp.float32),
        scratch_types=[
            pltpu.VMEM((CHUNK,), jnp.int32),
            pltpu.VMEM((CHUNK,), jnp.int32),
            pltpu.VMEM((CHUNK, d), jnp.float32),
            pltpu.VMEM((CHUNK, d), jnp.float32),
            pltpu.SemaphoreType.DMA,
            pltpu.SemaphoreType.DMA,
            pltpu.SemaphoreType.DMA,
            pltpu.SemaphoreType.DMA,
        ],
        compiler_params=pltpu.CompilerParams(use_tc_tiling_on_sc=False),
    )
    def k(idx_hbm, table_hbm, out_hbm, idx_v0, idx_v1, rows_v0, rows_v1,
          gsem0, gsem1, wsem0, wsem1):
        wid = lax.axis_index("s") * NUM_CORES + lax.axis_index("c")
        base = wid * b_per_w
        idx_v = (idx_v0, idx_v1)
        rows_v = (rows_v0, rows_v1)
        gsem = (gsem0, gsem1)
        wsem = (wsem0, wsem1)

        pltpu.sync_copy(idx_hbm.at[pl.ds(base, CHUNK)], idx_v[0])
        g0 = pltpu.async_copy(table_hbm.at[idx_v[0]], rows_v[0], gsem[0])

        def body(gi, carry):
            c0 = gi * 2
            for p in (0, 1):
                c = c0 + p
                q = p ^ 1
                @pl.when(c + 1 < nchunks)
                def _():
                    off_n = base + (c + 1) * CHUNK
                    @pl.when(c + 1 >= 2)
                    def _():
                        pltpu.make_async_copy(
                            rows_v[q], out_hbm.at[pl.ds(off_n, CHUNK)], wsem[q]
                        ).wait()
                    pltpu.sync_copy(idx_hbm.at[pl.ds(off_n, CHUNK)], idx_v[q])
                    pltpu.async_copy(table_hbm.at[idx_v[q]], rows_v[q], gsem[q])
                off = base + c * CHUNK
                pltpu.make_async_copy(
                    table_hbm.at[idx_v[p]], rows_v[p], gsem[p]
                ).wait()
                pltpu.async_copy(rows_v[p], out_hbm.at[pl.ds(off, CHUNK)], wsem[p])
            return carry

        lax.fori_loop(0, nchunks // 2, body, 0)
        off_last = base + (nchunks - 1) * CHUNK
        pltpu.make_async_copy(
            rows_v[0], out_hbm.at[pl.ds(base, CHUNK)], wsem[0]
        ).wait()
        pltpu.make_async_copy(
            rows_v[1], out_hbm.at[pl.ds(off_last, CHUNK)], wsem[1]
        ).wait()

    return k


def kernel(x, table):
    b0, b1 = x.shape
    total = b0 * b1
    xf = x.reshape(total).astype(jnp.int32)
    out = _gather_kernel(total, table.shape[1])(xf, table)
    return out.reshape(b0, b1, table.shape[1])
